# baseline (speedup 1.0000x reference)
import jax
import jax.numpy as jnp
from jax import lax
from jax.experimental import pallas as pl
from jax.experimental.pallas import tpu as pltpu
from jax.experimental.pallas import tpu_sc as plsc

N = 10000
D = 128
E = 320000
NTYPE = 5
SKEY = 65536
NC = 2
NS = 16
NW = NC * NS
BLK = 128
HBLK = 64
SCH = 8
SCHE = SCH * BLK
EP = 327680
SPW = EP // SCHE // NW
SPW0 = 20
HPAD = 10240
HROW = HPAD // NS
TBLK = 2000
f32 = jnp.float32
i32 = jnp.int32


def _tc_pre(nh_ref, wnf_ref, wa_ref, nf_ref, aa_ref):
    nhb = nh_ref[...]
    wnf = wnf_ref[...]
    nfb = jnp.dot(nhb, wnf.T, preferred_element_type=f32)
    nf_ref[...] = nfb
    wa = wa_ref[...]
    a_s = jnp.sum(nfb * wa[0:1, 0:D], axis=1, keepdims=True)
    a_d = jnp.sum(nfb * wa[0:1, D:2 * D], axis=1, keepdims=True)
    aa_ref[...] = jnp.concatenate([a_s, a_d], axis=1)


def _tc_post(nh_ref, h0_ref, h1_ref, wo_ref, b_ref, o_ref):
    nhb = nh_ref[...]
    hb = h0_ref[...] + h1_ref[...]
    wo = wo_ref[...]
    acc = jnp.dot(nhb, wo[:, 0:D].T, preferred_element_type=f32)
    acc = acc + jnp.dot(hb, wo[:, D:2 * D].T, preferred_element_type=f32)
    acc = acc + b_ref[...]
    o_ref[...] = jnp.maximum(acc, 0.0)


def _lane_bcast(v, rr):
    dn = lax.GatherDimensionNumbers(
        offset_dims=(), collapsed_slice_dims=(0,), start_index_map=(0,))
    return lax.gather(v, jnp.full((16, 1), rr, i32), dn, (1,),
                      mode=lax.GatherScatterMode.PROMISE_IN_BOUNDS)


def _sc_edge(aa_hbm, src_hbm, dst_hbm, y_hbm, e_hbm, s_hbm,
             aa_v, src_v, dst_v, y_v, e_v,
             k0, k1, k2, k3, k4, k5, k6, k7, z_v, s_sh, ssem):
    cid = lax.axis_index("c")
    sid = lax.axis_index("s")
    w = cid * NS + sid
    start = w * SPW * SCHE
    keyb = (k0, k1, k2, k3, k4, k5, k6, k7)

    pltpu.sync_copy(aa_hbm, aa_v)

    zch = SKEY // NS
    z16 = jnp.zeros((16,), f32)

    def zfill(i, _):
        z_v[pl.ds(i * 16, 16)] = z16
        return 0
    lax.fori_loop(0, zch // 16, zfill, 0)
    pltpu.sync_copy(z_v, s_sh.at[pl.ds(sid * zch, zch)])
    plsc.subcore_barrier()

    off_d = jnp.full((16,), N + 16, i32)

    def sc_body(sc, _):
        off = start + sc * SCHE
        pltpu.sync_copy(src_hbm.at[pl.ds(off, SCHE)], src_v)
        pltpu.sync_copy(dst_hbm.at[pl.ds(off, SCHE)], dst_v)
        pltpu.sync_copy(y_hbm.at[pl.ds(off, SCHE)], y_v)
        for b in range(SCH):
            for g in range(BLK // 16):
                o = b * BLK + g * 16
                sl = pl.ds(o, 16)
                s16 = src_v[sl]
                d16 = dst_v[sl]
                y16 = y_v[sl]
                a_s = plsc.load_gather(aa_v, [s16])
                a_d = plsc.load_gather(aa_v, [d16 + off_d])
                ef = a_s + a_d
                ef = jnp.where(ef > 0.0, ef, 0.2 * ef)
                e_v[sl] = jnp.exp(ef)
                keyb[b][pl.ds(g * 16, 16)] = d16 * NTYPE + y16
        pltpu.sync_copy(e_v, e_hbm.at[pl.ds(off, SCHE)])
        descs = []
        for b in range(SCH):
            descs.append(pltpu.async_copy(
                e_v.at[pl.ds(b * BLK, BLK)], s_sh.at[keyb[b]], ssem,
                add=True))
        for d in descs:
            d.wait()
        return 0
    lax.fori_loop(0, SPW, sc_body, 0)

    plsc.subcore_barrier()

    @pl.when(sid == 0)
    def _():
        pltpu.sync_copy(s_sh, s_hbm.at[cid])


def _sc_agg(nf_hbm, s_hbm, e_hbm, src_hbm, dst_hbm, y_hbm, h_hbm,
            t0_v, t1_v, src_v, dst_v, y_v, e_v, key_v, sval_v,
            r0, r1, r2, r3, df0, df1, df2, df3, z_v, s_sh, h_sh,
            g0, g1, g2, g3, s0, s1, s2, s3, vsem):
    cid = lax.axis_index("c")
    sid = lax.axis_index("s")
    nsc = jnp.where(cid == 0, SPW0, 2 * SPW - SPW0)
    start_sc = jnp.where(cid == 0, sid * SPW0,
                         NS * SPW0 + sid * (2 * SPW - SPW0))
    bufs = (r0, r1, r2, r3)
    dfb = (df0, df1, df2, df3)
    gsem = (g0, g1, g2, g3)
    ssem = (s0, s1, s2, s3)
    MPS = SCHE // HBLK

    tch = 2048

    def sbuild(hf, _):
        o = sid * (SKEY // NS) + hf * tch
        pltpu.sync_copy(s_hbm.at[0, pl.ds(o, tch)], t0_v)
        pltpu.sync_copy(s_hbm.at[1, pl.ds(o, tch)], t1_v)

        def add16(i, _):
            sl = pl.ds(i * 16, 16)
            t0_v[sl] = t0_v[sl] + t1_v[sl]
            return 0
        lax.fori_loop(0, tch // 16, add16, 0)
        pltpu.sync_copy(t0_v, s_sh.at[pl.ds(o, tch)])
        return 0
    lax.fori_loop(0, (SKEY // NS) // tch, sbuild, 0)

    z16 = jnp.zeros((16,), f32)

    def zfill(i, _):
        for j in range(D // 16):
            z_v[i, pl.ds(j * 16, 16)] = z16
        return 0
    lax.fori_loop(0, 8, zfill, 0)

    def zcopy(i, _):
        pltpu.sync_copy(z_v, h_sh.at[pl.ds(sid * HROW + i * 8, 8)])
        return 0
    lax.fori_loop(0, HROW // 8, zcopy, 0)
    plsc.subcore_barrier()

    def _gather(m, q):
        return pltpu.async_copy(
            nf_hbm.at[src_v.at[pl.ds(m * HBLK, HBLK)]], bufs[q], gsem[q])

    def _wait_gather(q):
        pltpu.make_async_copy(
            nf_hbm.at[pl.ds(0, HBLK)], bufs[q], gsem[q]).wait()

    def _wait_scatter(q):
        pltpu.make_async_copy(
            h_hbm.at[0, pl.ds(0, HBLK)], bufs[q], ssem[q]).wait()

    def sc_body(sc, _):
        off = (start_sc + sc) * SCHE
        pltpu.sync_copy(src_hbm.at[pl.ds(off, SCHE)], src_v)
        pltpu.sync_copy(dst_hbm.at[pl.ds(off, SCHE)], dst_v)
        pltpu.sync_copy(y_hbm.at[pl.ds(off, SCHE)], y_v)
        pltpu.sync_copy(e_hbm.at[pl.ds(off, SCHE)], e_v)

        def keys16(i, _):
            sl = pl.ds(i * 16, 16)
            key_v[sl] = dst_v[sl] * NTYPE + y_v[sl]
            return 0
        lax.fori_loop(0, SCHE // 16, keys16, 0)
        vdescs = [pltpu.async_copy(
            s_sh.at[key_v.at[pl.ds(b * BLK, BLK)]],
            sval_v.at[pl.ds(b * BLK, BLK)], vsem) for b in range(SCH)]
        for vd in vdescs:
            vd.wait()

        def alpha16(i, _):
            sl = pl.ds(i * 16, 16)
            e_v[sl] = e_v[sl] / sval_v[sl]
            return 0
        lax.fori_loop(0, SCHE // 16, alpha16, 0)

        for q in range(3):
            _gather(q, q)

        def uloop(u, _):
            for dq in range(4):
                m = 4 * u + dq
                nq = (dq + 3) % 4
                if dq == 0:
                    @pl.when(u >= 1)
                    def _():
                        _wait_scatter(nq)
                    _gather(m + 3, nq)
                else:
                    @pl.when(u < MPS // 4 - 1)
                    def _():
                        _wait_scatter(nq)
                        _gather(m + 3, nq)
                _wait_gather(dq)
                pltpu.sync_copy(
                    dst_hbm.at[pl.ds(off + m * HBLK, HBLK)], dfb[dq])

                def scale_g(g, _):
                    a16 = e_v[pl.ds(m * HBLK + g * 16, 16)]
                    for rr in range(16):
                        r = g * 16 + rr
                        arr = _lane_bcast(a16, rr)
                        for j in range(D // 16):
                            sj = pl.ds(j * 16, 16)
                            bufs[dq][r, sj] = bufs[dq][r, sj] * arr
                    return 0
                lax.fori_loop(0, HBLK // 16, scale_g, 0)
                pltpu.async_copy(bufs[dq], h_sh.at[dfb[dq]], ssem[dq],
                                 add=True)
            return 0
        lax.fori_loop(0, MPS // 4, uloop, 0)
        for q in range(4):
            _wait_scatter(q)
        return 0
    lax.fori_loop(0, nsc, sc_body, 0)

    plsc.subcore_barrier()

    nchunk = jnp.where(sid == NS - 1, 10, 16)

    def hout(i, _):
        r0_ = sid * HROW + i * 40
        pltpu.sync_copy(h_sh.at[pl.ds(r0_, 40)],
                        h_hbm.at[cid, pl.ds(r0_, 40)])
        return 0
    lax.fori_loop(0, nchunk, hout, 0)


def kernel(nh, W_nf, W_attn, W_out, b_out, edge_y, edge_index):
    nh = nh.astype(f32)
    src = edge_index[0].astype(i32)
    dst = edge_index[1].astype(i32)
    y = edge_y.astype(i32)
    npad = EP - E
    src = jnp.concatenate([src, jnp.zeros((npad,), i32)])
    dst = jnp.concatenate([dst, jnp.full((npad,), N, i32)])
    y = jnp.concatenate([y, jnp.zeros((npad,), i32)])

    nf, aa = pl.pallas_call(
        _tc_pre,
        grid=(N // TBLK,),
        in_specs=[
            pl.BlockSpec((TBLK, D), lambda i: (i, 0)),
            pl.BlockSpec((D, D), lambda i: (0, 0)),
            pl.BlockSpec((1, 2 * D), lambda i: (0, 0)),
        ],
        out_specs=[
            pl.BlockSpec((TBLK, D), lambda i: (i, 0)),
            pl.BlockSpec((TBLK, 2), lambda i: (i, 0)),
        ],
        out_shape=[
            jax.ShapeDtypeStruct((N, D), f32),
            jax.ShapeDtypeStruct((N, 2), f32),
        ],
    )(nh, W_nf, W_attn)

    z16f = jnp.zeros((16,), f32)
    aa = jnp.concatenate([aa[:, 0], z16f, aa[:, 1], z16f])

    mesh = plsc.VectorSubcoreMesh(core_axis_name="c", subcore_axis_name="s")
    sc_params = pltpu.CompilerParams(needs_layout_passes=False)
    kblk = [pltpu.VMEM((BLK,), i32) for _ in range(SCH)]

    e, s = pl.kernel(
        _sc_edge,
        out_type=[
            jax.ShapeDtypeStruct((EP,), f32),
            jax.ShapeDtypeStruct((NC, SKEY), f32),
        ],
        mesh=mesh,
        compiler_params=sc_params,
        scratch_types=[
            pltpu.VMEM((2 * (N + 16),), f32),
            pltpu.VMEM((SCHE,), i32),
            pltpu.VMEM((SCHE,), i32),
            pltpu.VMEM((SCHE,), i32),
            pltpu.VMEM((SCHE,), f32),
            *kblk,
            pltpu.VMEM((SKEY // NS,), f32),
            pltpu.VMEM_SHARED((SKEY,), f32),
            pltpu.SemaphoreType.DMA,
        ],
    )(aa, src, dst, y)

    h = pl.kernel(
        _sc_agg,
        out_type=jax.ShapeDtypeStruct((NC, HPAD, D), f32),
        mesh=mesh,
        compiler_params=sc_params,
        scratch_types=[
            pltpu.VMEM((2048,), f32),
            pltpu.VMEM((2048,), f32),
            pltpu.VMEM((SCHE,), i32),
            pltpu.VMEM((SCHE,), i32),
            pltpu.VMEM((SCHE,), i32),
            pltpu.VMEM((SCHE,), f32),
            pltpu.VMEM((SCHE,), i32),
            pltpu.VMEM((SCHE,), f32),
            pltpu.VMEM((HBLK, D), f32),
            pltpu.VMEM((HBLK, D), f32),
            pltpu.VMEM((HBLK, D), f32),
            pltpu.VMEM((HBLK, D), f32),
            pltpu.VMEM((HBLK,), i32),
            pltpu.VMEM((HBLK,), i32),
            pltpu.VMEM((HBLK,), i32),
            pltpu.VMEM((HBLK,), i32),
            pltpu.VMEM((8, D), f32),
            pltpu.VMEM_SHARED((SKEY,), f32),
            pltpu.VMEM_SHARED((HPAD, D), f32),
            pltpu.SemaphoreType.DMA,
            pltpu.SemaphoreType.DMA,
            pltpu.SemaphoreType.DMA,
            pltpu.SemaphoreType.DMA,
            pltpu.SemaphoreType.DMA,
            pltpu.SemaphoreType.DMA,
            pltpu.SemaphoreType.DMA,
            pltpu.SemaphoreType.DMA,
            pltpu.SemaphoreType.DMA,
        ],
    )(nf, s, e, src, dst, y)

    out = pl.pallas_call(
        _tc_post,
        grid=(N // TBLK,),
        in_specs=[
            pl.BlockSpec((TBLK, D), lambda i: (i, 0)),
            pl.BlockSpec((TBLK, D), lambda i: (i, 0)),
            pl.BlockSpec((TBLK, D), lambda i: (i, 0)),
            pl.BlockSpec((D, 2 * D), lambda i: (0, 0)),
            pl.BlockSpec((1, D), lambda i: (0, 0)),
        ],
        out_specs=pl.BlockSpec((TBLK, D), lambda i: (i, 0)),
        out_shape=jax.ShapeDtypeStruct((N, D), f32),
    )(nh, h[0, :N], h[1, :N], W_out, b_out.reshape(1, D))
    return out


# speedup vs baseline: 1.5302x; 1.5302x over previous
import jax
import jax.numpy as jnp
from jax import lax
from jax.experimental import pallas as pl
from jax.experimental.pallas import tpu as pltpu
from jax.experimental.pallas import tpu_sc as plsc

N = 10000
D = 128
E = 320000
NTYPE = 5
SKEY = 65536
NC = 2
NS = 16
NW = NC * NS
BLK = 128
HBLK = 64
SCH = 8
SCHE = SCH * BLK
EP = 327680
SPW = EP // SCHE // NW
SPW0 = 18
HPAD = 10240
HROW = HPAD // NS
TBLK = 2000
f32 = jnp.float32
i32 = jnp.int32


def _tc_pre(nh_ref, wnf_ref, wa_ref, nf_ref, aa_ref):
    nhb = nh_ref[...]
    wnf = wnf_ref[...]
    nfb = jnp.dot(nhb, wnf.T, preferred_element_type=f32)
    nf_ref[...] = nfb
    wa = wa_ref[...]
    a_s = jnp.sum(nfb * wa[0:1, 0:D], axis=1, keepdims=True)
    a_d = jnp.sum(nfb * wa[0:1, D:2 * D], axis=1, keepdims=True)
    aa_ref[...] = jnp.concatenate([a_s, a_d], axis=1)


def _tc_post(nh_ref, h0_ref, h1_ref, wo_ref, b_ref, o_ref):
    nhb = nh_ref[...]
    hb = h0_ref[...] + h1_ref[...]
    wo = wo_ref[...]
    acc = jnp.dot(nhb, wo[:, 0:D].T, preferred_element_type=f32)
    acc = acc + jnp.dot(hb, wo[:, D:2 * D].T, preferred_element_type=f32)
    acc = acc + b_ref[...]
    o_ref[...] = jnp.maximum(acc, 0.0)


def _lane_bcast(v, rr):
    dn = lax.GatherDimensionNumbers(
        offset_dims=(), collapsed_slice_dims=(0,), start_index_map=(0,))
    return lax.gather(v, jnp.full((16, 1), rr, i32), dn, (1,),
                      mode=lax.GatherScatterMode.PROMISE_IN_BOUNDS)


def _sc_edge(aa_hbm, src_hbm, dst_hbm, y_hbm, e_hbm, s_hbm,
             aa_v, src_v, dst_v, y_v, e_v,
             k0, k1, k2, k3, k4, k5, k6, k7, z_v, s_sh, ssem):
    cid = lax.axis_index("c")
    sid = lax.axis_index("s")
    w = cid * NS + sid
    start = w * SPW * SCHE
    keyb = (k0, k1, k2, k3, k4, k5, k6, k7)

    pltpu.sync_copy(aa_hbm, aa_v)

    zch = SKEY // NS
    z16 = jnp.zeros((16,), f32)

    def zfill(i, _):
        z_v[pl.ds(i * 16, 16)] = z16
        return 0
    lax.fori_loop(0, zch // 16, zfill, 0)
    pltpu.sync_copy(z_v, s_sh.at[pl.ds(sid * zch, zch)])
    plsc.subcore_barrier()

    off_d = jnp.full((16,), N + 16, i32)

    def sc_body(sc, _):
        off = start + sc * SCHE
        pltpu.sync_copy(src_hbm.at[pl.ds(off, SCHE)], src_v)
        pltpu.sync_copy(dst_hbm.at[pl.ds(off, SCHE)], dst_v)
        pltpu.sync_copy(y_hbm.at[pl.ds(off, SCHE)], y_v)
        for b in range(SCH):
            for g in range(BLK // 16):
                o = b * BLK + g * 16
                sl = pl.ds(o, 16)
                s16 = src_v[sl]
                d16 = dst_v[sl]
                y16 = y_v[sl]
                a_s = plsc.load_gather(aa_v, [s16])
                a_d = plsc.load_gather(aa_v, [d16 + off_d])
                ef = a_s + a_d
                ef = jnp.where(ef > 0.0, ef, 0.2 * ef)
                e_v[sl] = jnp.exp(ef)
                keyb[b][pl.ds(g * 16, 16)] = d16 * NTYPE + y16
        pltpu.sync_copy(e_v, e_hbm.at[pl.ds(off, SCHE)])
        descs = []
        for b in range(SCH):
            descs.append(pltpu.async_copy(
                e_v.at[pl.ds(b * BLK, BLK)], s_sh.at[keyb[b]], ssem,
                add=True))
        for d in descs:
            d.wait()
        return 0
    lax.fori_loop(0, SPW, sc_body, 0)

    plsc.subcore_barrier()

    @pl.when(sid == 0)
    def _():
        pltpu.sync_copy(s_sh, s_hbm.at[cid])


def _sc_agg(nf_hbm, s_hbm, e_hbm, src_hbm, dst_hbm, y_hbm, h_hbm,
            t0_v, t1_v, src_v, dst_v, y_v, e_v, key_v, sval_v,
            r0, r1, r2, r3, df0, df1, df2, df3, z_v, s_sh, h_sh,
            g0, g1, g2, g3, s0, s1, s2, s3, vsem):
    cid = lax.axis_index("c")
    sid = lax.axis_index("s")
    nsc = jnp.where(cid == 0, SPW0, 2 * SPW - SPW0)
    start_sc = jnp.where(cid == 0, sid * SPW0,
                         NS * SPW0 + sid * (2 * SPW - SPW0))
    bufs = (r0, r1, r2, r3)
    dfb = (df0, df1, df2, df3)
    gsem = (g0, g1, g2, g3)
    ssem = (s0, s1, s2, s3)
    MPS = SCHE // HBLK

    tch = 2048

    def sbuild(hf, _):
        o = sid * (SKEY // NS) + hf * tch
        pltpu.sync_copy(s_hbm.at[0, pl.ds(o, tch)], t0_v)
        pltpu.sync_copy(s_hbm.at[1, pl.ds(o, tch)], t1_v)

        def add16(i, _):
            sl = pl.ds(i * 16, 16)
            t0_v[sl] = t0_v[sl] + t1_v[sl]
            return 0
        lax.fori_loop(0, tch // 16, add16, 0)
        pltpu.sync_copy(t0_v, s_sh.at[pl.ds(o, tch)])
        return 0
    lax.fori_loop(0, (SKEY // NS) // tch, sbuild, 0)

    z16 = jnp.zeros((16,), f32)

    def zfill(i, _):
        for j in range(D // 16):
            z_v[i, pl.ds(j * 16, 16)] = z16
        return 0
    lax.fori_loop(0, 8, zfill, 0)

    def zcopy(i, _):
        pltpu.sync_copy(z_v, h_sh.at[pl.ds(sid * HROW + i * 8, 8)])
        return 0
    lax.fori_loop(0, HROW // 8, zcopy, 0)
    plsc.subcore_barrier()

    def _gather(m, q):
        return pltpu.async_copy(
            nf_hbm.at[src_v.at[pl.ds(m * HBLK, HBLK)]], bufs[q], gsem[q])

    def _wait_gather(q):
        pltpu.make_async_copy(
            nf_hbm.at[pl.ds(0, HBLK)], bufs[q], gsem[q]).wait()

    def _wait_scatter(q):
        pltpu.make_async_copy(
            h_hbm.at[0, pl.ds(0, HBLK)], bufs[q], ssem[q]).wait()

    def sc_body(sc, _):
        off = (start_sc + sc) * SCHE
        pltpu.sync_copy(src_hbm.at[pl.ds(off, SCHE)], src_v)
        pltpu.sync_copy(dst_hbm.at[pl.ds(off, SCHE)], dst_v)
        pltpu.sync_copy(y_hbm.at[pl.ds(off, SCHE)], y_v)
        pltpu.sync_copy(e_hbm.at[pl.ds(off, SCHE)], e_v)

        def keys16(i, _):
            sl = pl.ds(i * 16, 16)
            key_v[sl] = dst_v[sl] * NTYPE + y_v[sl]
            return 0
        lax.fori_loop(0, SCHE // 16, keys16, 0)
        vdescs = [pltpu.async_copy(
            s_sh.at[key_v.at[pl.ds(b * BLK, BLK)]],
            sval_v.at[pl.ds(b * BLK, BLK)], vsem) for b in range(SCH)]
        for vd in vdescs:
            vd.wait()

        def alpha16(i, _):
            sl = pl.ds(i * 16, 16)
            e_v[sl] = e_v[sl] / sval_v[sl]
            return 0
        lax.fori_loop(0, SCHE // 16, alpha16, 0)

        for q in range(3):
            _gather(q, q)

        def uloop(u, _):
            for dq in range(4):
                m = 4 * u + dq
                nq = (dq + 3) % 4
                if dq == 0:
                    @pl.when(u >= 1)
                    def _():
                        _wait_scatter(nq)
                    _gather(m + 3, nq)
                else:
                    @pl.when(u < MPS // 4 - 1)
                    def _():
                        _wait_scatter(nq)
                        _gather(m + 3, nq)
                _wait_gather(dq)
                pltpu.sync_copy(
                    dst_hbm.at[pl.ds(off + m * HBLK, HBLK)], dfb[dq])

                def scale_g(g, _):
                    a16 = e_v[pl.ds(m * HBLK + g * 16, 16)]
                    for rr in range(16):
                        r = g * 16 + rr
                        arr = _lane_bcast(a16, rr)
                        for j in range(D // 16):
                            sj = pl.ds(j * 16, 16)
                            bufs[dq][r, sj] = bufs[dq][r, sj] * arr
                    return 0
                lax.fori_loop(0, HBLK // 16, scale_g, 0)
                pltpu.async_copy(bufs[dq], h_sh.at[dfb[dq]], ssem[dq],
                                 add=True)
            return 0
        lax.fori_loop(0, MPS // 4, uloop, 0)
        for q in range(4):
            _wait_scatter(q)
        return 0
    lax.fori_loop(0, nsc, sc_body, 0)

    plsc.subcore_barrier()

    nchunk = jnp.where(sid == NS - 1, 10, 16)

    def hout(i, _):
        r0_ = sid * HROW + i * 40
        pltpu.sync_copy(h_sh.at[pl.ds(r0_, 40)],
                        h_hbm.at[cid, pl.ds(r0_, 40)])
        return 0
    lax.fori_loop(0, nchunk, hout, 0)


def kernel(nh, W_nf, W_attn, W_out, b_out, edge_y, edge_index):
    nh = nh.astype(f32)
    src = edge_index[0].astype(i32)
    dst = edge_index[1].astype(i32)
    y = edge_y.astype(i32)
    npad = EP - E
    src = jnp.concatenate([src, jnp.zeros((npad,), i32)])
    dst = jnp.concatenate([dst, jnp.full((npad,), N, i32)])
    y = jnp.concatenate([y, jnp.zeros((npad,), i32)])

    nf, aa = pl.pallas_call(
        _tc_pre,
        grid=(N // TBLK,),
        in_specs=[
            pl.BlockSpec((TBLK, D), lambda i: (i, 0)),
            pl.BlockSpec((D, D), lambda i: (0, 0)),
            pl.BlockSpec((1, 2 * D), lambda i: (0, 0)),
        ],
        out_specs=[
            pl.BlockSpec((TBLK, D), lambda i: (i, 0)),
            pl.BlockSpec((TBLK, 2), lambda i: (i, 0)),
        ],
        out_shape=[
            jax.ShapeDtypeStruct((N, D), f32),
            jax.ShapeDtypeStruct((N, 2), f32),
        ],
    )(nh, W_nf, W_attn)

    z16f = jnp.zeros((16,), f32)
    aa = jnp.concatenate([aa[:, 0], z16f, aa[:, 1], z16f])

    mesh = plsc.VectorSubcoreMesh(core_axis_name="c", subcore_axis_name="s")
    sc_params = pltpu.CompilerParams(needs_layout_passes=False)
    kblk = [pltpu.VMEM((BLK,), i32) for _ in range(SCH)]

    e, s = pl.kernel(
        _sc_edge,
        out_type=[
            jax.ShapeDtypeStruct((EP,), f32),
            jax.ShapeDtypeStruct((NC, SKEY), f32),
        ],
        mesh=mesh,
        compiler_params=sc_params,
        scratch_types=[
            pltpu.VMEM((2 * (N + 16),), f32),
            pltpu.VMEM((SCHE,), i32),
            pltpu.VMEM((SCHE,), i32),
            pltpu.VMEM((SCHE,), i32),
            pltpu.VMEM((SCHE,), f32),
            *kblk,
            pltpu.VMEM((SKEY // NS,), f32),
            pltpu.VMEM_SHARED((SKEY,), f32),
            pltpu.SemaphoreType.DMA,
        ],
    )(aa, src, dst, y)

    h = pl.kernel(
        _sc_agg,
        out_type=jax.ShapeDtypeStruct((NC, HPAD, D), f32),
        mesh=mesh,
        compiler_params=sc_params,
        scratch_types=[
            pltpu.VMEM((2048,), f32),
            pltpu.VMEM((2048,), f32),
            pltpu.VMEM((SCHE,), i32),
            pltpu.VMEM((SCHE,), i32),
            pltpu.VMEM((SCHE,), i32),
            pltpu.VMEM((SCHE,), f32),
            pltpu.VMEM((SCHE,), i32),
            pltpu.VMEM((SCHE,), f32),
            pltpu.VMEM((HBLK, D), f32),
            pltpu.VMEM((HBLK, D), f32),
            pltpu.VMEM((HBLK, D), f32),
            pltpu.VMEM((HBLK, D), f32),
            pltpu.VMEM((HBLK,), i32),
            pltpu.VMEM((HBLK,), i32),
            pltpu.VMEM((HBLK,), i32),
            pltpu.VMEM((HBLK,), i32),
            pltpu.VMEM((8, D), f32),
            pltpu.VMEM_SHARED((SKEY,), f32),
            pltpu.VMEM_SHARED((HPAD, D), f32),
            pltpu.SemaphoreType.DMA,
            pltpu.SemaphoreType.DMA,
            pltpu.SemaphoreType.DMA,
            pltpu.SemaphoreType.DMA,
            pltpu.SemaphoreType.DMA,
            pltpu.SemaphoreType.DMA,
            pltpu.SemaphoreType.DMA,
            pltpu.SemaphoreType.DMA,
            pltpu.SemaphoreType.DMA,
        ],
    )(nf, s, e, src, dst, y)

    out = pl.pallas_call(
        _tc_post,
        grid=(N // TBLK,),
        in_specs=[
            pl.BlockSpec((TBLK, D), lambda i: (i, 0)),
            pl.BlockSpec((TBLK, D), lambda i: (i, 0)),
            pl.BlockSpec((TBLK, D), lambda i: (i, 0)),
            pl.BlockSpec((D, 2 * D), lambda i: (0, 0)),
            pl.BlockSpec((1, D), lambda i: (0, 0)),
        ],
        out_specs=pl.BlockSpec((TBLK, D), lambda i: (i, 0)),
        out_shape=jax.ShapeDtypeStruct((N, D), f32),
    )(nh, h[0, :N], h[1, :N], W_out, b_out.reshape(1, D))
    return out


# speedup vs baseline: 1.5588x; 1.0187x over previous
import jax
import jax.numpy as jnp
from jax import lax
from jax.experimental import pallas as pl
from jax.experimental.pallas import tpu as pltpu
from jax.experimental.pallas import tpu_sc as plsc

N = 10000
D = 128
E = 320000
NTYPE = 5
SKEY = 65536
NC = 2
NS = 16
NW = NC * NS
BLK = 128
HBLK = 64
SCH = 8
SCHE = SCH * BLK
EP = 327680
SPW = EP // SCHE // NW
SPW0 = 17
HPAD = 10240
HROW = HPAD // NS
TBLK = 2000
f32 = jnp.float32
i32 = jnp.int32


def _tc_pre(nh_ref, wnf_ref, wa_ref, nf_ref, aa_ref):
    nhb = nh_ref[...]
    wnf = wnf_ref[...]
    nfb = jnp.dot(nhb, wnf.T, preferred_element_type=f32)
    nf_ref[...] = nfb
    wa = wa_ref[...]
    a_s = jnp.sum(nfb * wa[0:1, 0:D], axis=1, keepdims=True)
    a_d = jnp.sum(nfb * wa[0:1, D:2 * D], axis=1, keepdims=True)
    aa_ref[...] = jnp.concatenate([a_s, a_d], axis=1)


def _tc_post(nh_ref, h0_ref, h1_ref, wo_ref, b_ref, o_ref):
    nhb = nh_ref[...]
    hb = h0_ref[...] + h1_ref[...]
    wo = wo_ref[...]
    acc = jnp.dot(nhb, wo[:, 0:D].T, preferred_element_type=f32)
    acc = acc + jnp.dot(hb, wo[:, D:2 * D].T, preferred_element_type=f32)
    acc = acc + b_ref[...]
    o_ref[...] = jnp.maximum(acc, 0.0)


def _lane_bcast(v, rr):
    dn = lax.GatherDimensionNumbers(
        offset_dims=(), collapsed_slice_dims=(0,), start_index_map=(0,))
    return lax.gather(v, jnp.full((16, 1), rr, i32), dn, (1,),
                      mode=lax.GatherScatterMode.PROMISE_IN_BOUNDS)


def _sc_edge(aa_hbm, src_hbm, dst_hbm, y_hbm, e_hbm, s_hbm,
             aa_v, src_v, dst_v, y_v, e_v,
             k0, k1, k2, k3, k4, k5, k6, k7, z_v, s_sh, ssem):
    cid = lax.axis_index("c")
    sid = lax.axis_index("s")
    w = cid * NS + sid
    start = w * SPW * SCHE
    keyb = (k0, k1, k2, k3, k4, k5, k6, k7)

    pltpu.sync_copy(aa_hbm, aa_v)

    zch = SKEY // NS
    z16 = jnp.zeros((16,), f32)

    def zfill(i, _):
        z_v[pl.ds(i * 16, 16)] = z16
        return 0
    lax.fori_loop(0, zch // 16, zfill, 0)
    pltpu.sync_copy(z_v, s_sh.at[pl.ds(sid * zch, zch)])
    plsc.subcore_barrier()

    off_d = jnp.full((16,), N + 16, i32)

    def sc_body(sc, _):
        off = start + sc * SCHE
        pltpu.sync_copy(src_hbm.at[pl.ds(off, SCHE)], src_v)
        pltpu.sync_copy(dst_hbm.at[pl.ds(off, SCHE)], dst_v)
        pltpu.sync_copy(y_hbm.at[pl.ds(off, SCHE)], y_v)
        for b in range(SCH):
            for g in range(BLK // 16):
                o = b * BLK + g * 16
                sl = pl.ds(o, 16)
                s16 = src_v[sl]
                d16 = dst_v[sl]
                y16 = y_v[sl]
                a_s = plsc.load_gather(aa_v, [s16])
                a_d = plsc.load_gather(aa_v, [d16 + off_d])
                ef = a_s + a_d
                ef = jnp.where(ef > 0.0, ef, 0.2 * ef)
                e_v[sl] = jnp.exp(ef)
                keyb[b][pl.ds(g * 16, 16)] = d16 * NTYPE + y16
        pltpu.sync_copy(e_v, e_hbm.at[pl.ds(off, SCHE)])
        descs = []
        for b in range(SCH):
            descs.append(pltpu.async_copy(
                e_v.at[pl.ds(b * BLK, BLK)], s_sh.at[keyb[b]], ssem,
                add=True))
        for d in descs:
            d.wait()
        return 0
    lax.fori_loop(0, SPW, sc_body, 0)

    plsc.subcore_barrier()

    @pl.when(sid == 0)
    def _():
        pltpu.sync_copy(s_sh, s_hbm.at[cid])


def _sc_agg(nf_hbm, s_hbm, e_hbm, src_hbm, dst_hbm, y_hbm, h_hbm,
            t0_v, t1_v, src_v, dst_v, y_v, e_v, key_v, sval_v,
            r0, r1, r2, r3, df0, df1, df2, df3, z_v, s_sh, h_sh,
            g0, g1, g2, g3, s0, s1, s2, s3, vsem):
    cid = lax.axis_index("c")
    sid = lax.axis_index("s")
    nsc = jnp.where(cid == 0, SPW0, 2 * SPW - SPW0)
    start_sc = jnp.where(cid == 0, sid * SPW0,
                         NS * SPW0 + sid * (2 * SPW - SPW0))
    bufs = (r0, r1, r2, r3)
    dfb = (df0, df1, df2, df3)
    gsem = (g0, g1, g2, g3)
    ssem = (s0, s1, s2, s3)
    MPS = SCHE // HBLK

    tch = 2048

    def sbuild(hf, _):
        o = sid * (SKEY // NS) + hf * tch
        pltpu.sync_copy(s_hbm.at[0, pl.ds(o, tch)], t0_v)
        pltpu.sync_copy(s_hbm.at[1, pl.ds(o, tch)], t1_v)

        def add16(i, _):
            sl = pl.ds(i * 16, 16)
            t0_v[sl] = t0_v[sl] + t1_v[sl]
            return 0
        lax.fori_loop(0, tch // 16, add16, 0)
        pltpu.sync_copy(t0_v, s_sh.at[pl.ds(o, tch)])
        return 0
    lax.fori_loop(0, (SKEY // NS) // tch, sbuild, 0)

    z16 = jnp.zeros((16,), f32)

    def zfill(i, _):
        for j in range(D // 16):
            z_v[i, pl.ds(j * 16, 16)] = z16
        return 0
    lax.fori_loop(0, 8, zfill, 0)

    def zcopy(i, _):
        pltpu.sync_copy(z_v, h_sh.at[pl.ds(sid * HROW + i * 8, 8)])
        return 0
    lax.fori_loop(0, HROW // 8, zcopy, 0)
    plsc.subcore_barrier()

    def _gather(m, q):
        return pltpu.async_copy(
            nf_hbm.at[src_v.at[pl.ds(m * HBLK, HBLK)]], bufs[q], gsem[q])

    def _wait_gather(q):
        pltpu.make_async_copy(
            nf_hbm.at[pl.ds(0, HBLK)], bufs[q], gsem[q]).wait()

    def _wait_scatter(q):
        pltpu.make_async_copy(
            h_hbm.at[0, pl.ds(0, HBLK)], bufs[q], ssem[q]).wait()

    def sc_body(sc, _):
        off = (start_sc + sc) * SCHE
        pltpu.sync_copy(src_hbm.at[pl.ds(off, SCHE)], src_v)
        pltpu.sync_copy(dst_hbm.at[pl.ds(off, SCHE)], dst_v)
        pltpu.sync_copy(y_hbm.at[pl.ds(off, SCHE)], y_v)
        pltpu.sync_copy(e_hbm.at[pl.ds(off, SCHE)], e_v)

        def keys16(i, _):
            sl = pl.ds(i * 16, 16)
            key_v[sl] = dst_v[sl] * NTYPE + y_v[sl]
            return 0
        lax.fori_loop(0, SCHE // 16, keys16, 0)
        vdescs = [pltpu.async_copy(
            s_sh.at[key_v.at[pl.ds(b * BLK, BLK)]],
            sval_v.at[pl.ds(b * BLK, BLK)], vsem) for b in range(SCH)]
        for vd in vdescs:
            vd.wait()

        def alpha16(i, _):
            sl = pl.ds(i * 16, 16)
            e_v[sl] = e_v[sl] / sval_v[sl]
            return 0
        lax.fori_loop(0, SCHE // 16, alpha16, 0)

        for q in range(3):
            _gather(q, q)

        def uloop(u, _):
            for dq in range(4):
                m = 4 * u + dq
                nq = (dq + 3) % 4
                if dq == 0:
                    @pl.when(u >= 1)
                    def _():
                        _wait_scatter(nq)
                    _gather(m + 3, nq)
                else:
                    @pl.when(u < MPS // 4 - 1)
                    def _():
                        _wait_scatter(nq)
                        _gather(m + 3, nq)
                _wait_gather(dq)
                pltpu.sync_copy(
                    dst_hbm.at[pl.ds(off + m * HBLK, HBLK)], dfb[dq])

                def scale_g(g, _):
                    a16 = e_v[pl.ds(m * HBLK + g * 16, 16)]
                    for rr in range(16):
                        r = g * 16 + rr
                        arr = _lane_bcast(a16, rr)
                        for j in range(D // 16):
                            sj = pl.ds(j * 16, 16)
                            bufs[dq][r, sj] = bufs[dq][r, sj] * arr
                    return 0
                lax.fori_loop(0, HBLK // 16, scale_g, 0)
                pltpu.async_copy(bufs[dq], h_sh.at[dfb[dq]], ssem[dq],
                                 add=True)
            return 0
        lax.fori_loop(0, MPS // 4, uloop, 0)
        for q in range(4):
            _wait_scatter(q)
        return 0
    lax.fori_loop(0, nsc, sc_body, 0)

    plsc.subcore_barrier()

    nchunk = jnp.where(sid == NS - 1, 10, 16)

    def hout(i, _):
        r0_ = sid * HROW + i * 40
        pltpu.sync_copy(h_sh.at[pl.ds(r0_, 40)],
                        h_hbm.at[cid, pl.ds(r0_, 40)])
        return 0
    lax.fori_loop(0, nchunk, hout, 0)


def kernel(nh, W_nf, W_attn, W_out, b_out, edge_y, edge_index):
    nh = nh.astype(f32)
    src = edge_index[0].astype(i32)
    dst = edge_index[1].astype(i32)
    y = edge_y.astype(i32)
    npad = EP - E
    src = jnp.concatenate([src, jnp.zeros((npad,), i32)])
    dst = jnp.concatenate([dst, jnp.full((npad,), N, i32)])
    y = jnp.concatenate([y, jnp.zeros((npad,), i32)])

    nf, aa = pl.pallas_call(
        _tc_pre,
        grid=(N // TBLK,),
        in_specs=[
            pl.BlockSpec((TBLK, D), lambda i: (i, 0)),
            pl.BlockSpec((D, D), lambda i: (0, 0)),
            pl.BlockSpec((1, 2 * D), lambda i: (0, 0)),
        ],
        out_specs=[
            pl.BlockSpec((TBLK, D), lambda i: (i, 0)),
            pl.BlockSpec((TBLK, 2), lambda i: (i, 0)),
        ],
        out_shape=[
            jax.ShapeDtypeStruct((N, D), f32),
            jax.ShapeDtypeStruct((N, 2), f32),
        ],
    )(nh, W_nf, W_attn)

    z16f = jnp.zeros((16,), f32)
    aa = jnp.concatenate([aa[:, 0], z16f, aa[:, 1], z16f])

    mesh = plsc.VectorSubcoreMesh(core_axis_name="c", subcore_axis_name="s")
    sc_params = pltpu.CompilerParams(needs_layout_passes=False)
    kblk = [pltpu.VMEM((BLK,), i32) for _ in range(SCH)]

    e, s = pl.kernel(
        _sc_edge,
        out_type=[
            jax.ShapeDtypeStruct((EP,), f32),
            jax.ShapeDtypeStruct((NC, SKEY), f32),
        ],
        mesh=mesh,
        compiler_params=sc_params,
        scratch_types=[
            pltpu.VMEM((2 * (N + 16),), f32),
            pltpu.VMEM((SCHE,), i32),
            pltpu.VMEM((SCHE,), i32),
            pltpu.VMEM((SCHE,), i32),
            pltpu.VMEM((SCHE,), f32),
            *kblk,
            pltpu.VMEM((SKEY // NS,), f32),
            pltpu.VMEM_SHARED((SKEY,), f32),
            pltpu.SemaphoreType.DMA,
        ],
    )(aa, src, dst, y)

    h = pl.kernel(
        _sc_agg,
        out_type=jax.ShapeDtypeStruct((NC, HPAD, D), f32),
        mesh=mesh,
        compiler_params=sc_params,
        scratch_types=[
            pltpu.VMEM((2048,), f32),
            pltpu.VMEM((2048,), f32),
            pltpu.VMEM((SCHE,), i32),
            pltpu.VMEM((SCHE,), i32),
            pltpu.VMEM((SCHE,), i32),
            pltpu.VMEM((SCHE,), f32),
            pltpu.VMEM((SCHE,), i32),
            pltpu.VMEM((SCHE,), f32),
            pltpu.VMEM((HBLK, D), f32),
            pltpu.VMEM((HBLK, D), f32),
            pltpu.VMEM((HBLK, D), f32),
            pltpu.VMEM((HBLK, D), f32),
            pltpu.VMEM((HBLK,), i32),
            pltpu.VMEM((HBLK,), i32),
            pltpu.VMEM((HBLK,), i32),
            pltpu.VMEM((HBLK,), i32),
            pltpu.VMEM((8, D), f32),
            pltpu.VMEM_SHARED((SKEY,), f32),
            pltpu.VMEM_SHARED((HPAD, D), f32),
            pltpu.SemaphoreType.DMA,
            pltpu.SemaphoreType.DMA,
            pltpu.SemaphoreType.DMA,
            pltpu.SemaphoreType.DMA,
            pltpu.SemaphoreType.DMA,
            pltpu.SemaphoreType.DMA,
            pltpu.SemaphoreType.DMA,
            pltpu.SemaphoreType.DMA,
            pltpu.SemaphoreType.DMA,
        ],
    )(nf, s, e, src, dst, y)

    out = pl.pallas_call(
        _tc_post,
        grid=(N // TBLK,),
        in_specs=[
            pl.BlockSpec((TBLK, D), lambda i: (i, 0)),
            pl.BlockSpec((TBLK, D), lambda i: (i, 0)),
            pl.BlockSpec((TBLK, D), lambda i: (i, 0)),
            pl.BlockSpec((D, 2 * D), lambda i: (0, 0)),
            pl.BlockSpec((1, D), lambda i: (0, 0)),
        ],
        out_specs=pl.BlockSpec((TBLK, D), lambda i: (i, 0)),
        out_shape=jax.ShapeDtypeStruct((N, D), f32),
    )(nh, h[0, :N], h[1, :N], W_out, b_out.reshape(1, D))
    return out
